# baseline (speedup 1.0000x reference)
import functools
import jax
import jax.numpy as jnp
from jax import lax
from jax.experimental import pallas as pl
from jax.experimental.pallas import tpu as pltpu
from jax.experimental.pallas import tpu_sc as plsc

B, H, W = 1024, 30, 30
P = H * W
NV, DV, DP, D = 10, 64, 64, 128
WPAD = 32
NG = B * H
NP = NG * WPAD
NC, NS = 2, 16
NW = NC * NS
NPW = NP // NW
CH = 128
NCH = NPW // CH
NBUF = 3
NOUT = NCH // NBUF


def _sc_body(fused_hbm, g_hbm, poff_hbm, out_hbm, gvm, pvm, rb0, rb1, rb2,
             s0, s1, s2):
    rbufs = (rb0, rb1, rb2)
    sems = (s0, s1, s2)
    wid = lax.axis_index("s") * NC + lax.axis_index("c")
    pltpu.sync_copy(g_hbm.at[wid], gvm)
    pltpu.sync_copy(poff_hbm, pvm)

    def add_body(c, carry):
        for j in range(8):
            sl = pl.ds(j * 16, 16)
            gvm[c, sl] = gvm[c, sl] + pvm[c, sl]
        return carry
    lax.fori_loop(0, NCH, add_body, 0)

    def gather_start(c, b):
        pltpu.async_copy(fused_hbm.at[gvm.at[c]], rbufs[b], sems[b])

    def chunk_done(c, b):
        pltpu.make_async_copy(fused_hbm.at[gvm.at[c]], rbufs[b], sems[b]).wait()
        pltpu.sync_copy(rbufs[b], out_hbm.at[wid, pl.ds(c * CH, CH)])

    for b in range(NBUF):
        gather_start(b, b)

    def outer(o, carry):
        for b in range(NBUF):
            c = o * NBUF + b
            chunk_done(c, b)
            gather_start(c + NBUF, b)
        return carry
    lax.fori_loop(0, NOUT - 1, outer, 0)
    for b in range(NBUF):
        chunk_done((NOUT - 1) * NBUF + b, b)


_sc_call = functools.partial(
    pl.kernel,
    out_type=jax.ShapeDtypeStruct((NW, NPW, D), jnp.float32),
    mesh=plsc.VectorSubcoreMesh(core_axis_name="c", subcore_axis_name="s"),
    scratch_types=[
        pltpu.VMEM((NCH, CH), jnp.int32),
        pltpu.VMEM((NCH, CH), jnp.int32),
        pltpu.VMEM((CH, D), jnp.float32),
        pltpu.VMEM((CH, D), jnp.float32),
        pltpu.VMEM((CH, D), jnp.float32),
        pltpu.SemaphoreType.DMA,
        pltpu.SemaphoreType.DMA,
        pltpu.SemaphoreType.DMA,
    ],
)(_sc_body)


def _pad_groups(x):
    xp = jnp.pad(x.reshape(NG, W), ((0, 0), (0, WPAD - W)))
    return xp.reshape(NW, NCH, CH)


def kernel(grid, value_embed, pos_encoding):
    gflat = grid.astype(jnp.int32).reshape(B * P)
    g3 = _pad_groups(gflat)
    poff = _pad_groups((jnp.arange(B * P, dtype=jnp.int32) % P) * NV)[0]
    pos2 = pos_encoding.reshape(P, DP)
    fused = jnp.concatenate(
        [jnp.broadcast_to(value_embed[None], (P, NV, DV)),
         jnp.broadcast_to(pos2[:, None, :], (P, NV, DP))],
        axis=-1).reshape(P * NV, D)
    out = _sc_call(fused, g3, poff)
    return out.reshape(B, H, WPAD, D)[:, :, :W, :]


# speedup vs baseline: 6.6677x; 6.6677x over previous
import jax
import jax.numpy as jnp
from jax.experimental import pallas as pl
from jax.experimental.pallas import tpu as pltpu

B, H, W = 1024, 30, 30
NV, DV = 10, 64
DP = 64
D = 128
BB = 32


def _embed_body(g_ref, ve_ref, pos_ref, out_ref):
    x3 = jnp.broadcast_to(ve_ref[0:8][None], (BB, 8, DV))
    row8 = ve_ref[8][None, None, :]
    row9 = ve_ref[9][None, None, :]
    for h in range(H):
        idx3 = jnp.broadcast_to(g_ref[:, h][..., None], (BB, W, DV))
        val = jnp.take_along_axis(x3, idx3 & 7, axis=1,
                                  mode="promise_in_bounds")
        val = jnp.where(idx3 == 8, row8, val)
        val = jnp.where(idx3 == 9, row9, val)
        pos_h = jnp.broadcast_to(pos_ref[h][None], (BB, W, DP))
        out_ref[:, h] = jnp.concatenate([val, pos_h], axis=-1)


def kernel(grid, value_embed, pos_encoding):
    g32 = grid.astype(jnp.int32)
    out = pl.pallas_call(
        _embed_body,
        grid=(B // BB,),
        in_specs=[
            pl.BlockSpec((BB, H, W), lambda i: (i, 0, 0)),
            pl.BlockSpec((NV, DV), lambda i: (0, 0)),
            pl.BlockSpec((H, W, DP), lambda i: (0, 0, 0)),
        ],
        out_specs=pl.BlockSpec((BB, H, W, D), lambda i: (i, 0, 0, 0)),
        out_shape=jax.ShapeDtypeStruct((B, H, W, D), jnp.float32),
    )(g32, value_embed, pos_encoding)
    return out
